# baseline (speedup 1.0000x reference)
import functools

import jax
import jax.numpy as jnp
from jax import lax
from jax.experimental import pallas as pl
from jax.experimental.pallas import tpu as pltpu
from jax.experimental.pallas import tpu_sc as plsc

NUM_CHARS = 1000
EMB_DIM = 64
_VPAD = 1024
_NCHUNK = 8
_BATCH = 4096
_SEQ = 20
_NVT = NUM_CHARS // 8


def _mm_body(emb_ref, w_ref, b_ref, mc_ref):
    mc_ref[0] = lax.dot_general(
        emb_ref[...], w_ref[0],
        dimension_numbers=(((1,), (1,)), ((), ())),
        preferred_element_type=jnp.float32,
    ) + b_ref[0]


def _make_table(emb_table, W, b):
    w_pad = jnp.zeros((_VPAD, EMB_DIM), jnp.float32).at[:NUM_CHARS].set(W)
    b_pad = jnp.zeros((_VPAD,), jnp.float32).at[:NUM_CHARS].set(b)
    return pl.pallas_call(
        _mm_body,
        grid=(_NCHUNK,),
        in_specs=[
            pl.BlockSpec((NUM_CHARS, EMB_DIM), lambda c: (0, 0)),
            pl.BlockSpec((1, 128, EMB_DIM), lambda c: (c, 0, 0)),
            pl.BlockSpec((1, 1, 128), lambda c: (c, 0, 0)),
        ],
        out_specs=pl.BlockSpec((1, NUM_CHARS, 128), lambda c: (c, 0, 0)),
        out_shape=jax.ShapeDtypeStruct((_NCHUNK, NUM_CHARS, 128), jnp.float32),
    )(emb_table, w_pad.reshape(_NCHUNK, 128, EMB_DIM),
      b_pad.reshape(_NCHUNK, 1, 128))


_NC, _NS = 2, 16
_NW = _NC * _NS
_NBT = _BATCH // 128
_NBLK = _SEQ * _NBT
_BLK_W = _NBLK // _NW
_TOK_W = _BLK_W * 128


def _sc_body(mc_hbm, xt_hbm, out_hbm, idx_v, g0, g1, t0, t1,
             sg0, sg1, sw0, sw1):
    wid = lax.axis_index("s") * _NC + lax.axis_index("c")
    tbase = wid * _TOK_W
    pltpu.sync_copy(xt_hbm.at[pl.ds(tbase, _TOK_W)], idx_v)

    G = (g0, g1)
    T = (t0, t1)
    SG = (sg0, sg1)
    SW = (sw0, sw1)
    iota = lax.iota(jnp.int32, 16)

    def g_desc(par, i_blk, c):
        return pltpu.make_async_copy(
            mc_hbm.at[c].at[idx_v.at[pl.ds(i_blk * 128, 128)]], G[par],
            SG[par])

    def w_desc(par, i_blk, c):
        blockid = wid * _BLK_W + i_blk
        s = blockid // _NBT
        bt = blockid % _NBT
        nvt = 16 if c < 7 else 13
        return pltpu.make_async_copy(
            T[par].at[pl.ds(0, nvt)],
            out_hbm.at[pl.ds(s * _NVT + 16 * c, nvt), bt],
            SW[par])

    def run_task(i_blk, c):
        par = c % 2
        g_desc(par, i_blk, c).wait()
        if c < 7:
            g_desc(1 - par, i_blk, c + 1).start()
        else:
            @pl.when(i_blk < _BLK_W - 1)
            def _():
                g_desc(1 - par, i_blk + 1, 0).start()
        if c >= 2:
            w_desc(par, i_blk, c - 2).wait()
        else:
            @pl.when(i_blk > 0)
            def _():
                w_desc(par, i_blk - 1, c + 6).wait()
        nv = 128 if c < 7 else 104

        def trans_v(v, carry):
            row = v // 8
            col0 = (v % 8) * 128
            for g in range(8):
                vals = plsc.load_gather(
                    G[par], [iota + (16 * g), jnp.full((16,), v, jnp.int32)])
                T[par][row, pl.ds(col0 + 16 * g, 16)] = vals
            return carry

        lax.fori_loop(0, nv, trans_v, 0)
        w_desc(par, i_blk, c).start()

    g_desc(0, 0, 0).start()

    def blk(i_blk, carry):
        for c in range(_NCHUNK):
            run_task(i_blk, c)
        return carry

    lax.fori_loop(0, _BLK_W, blk, 0)
    w_desc(0, _BLK_W - 1, 6).wait()
    w_desc(1, _BLK_W - 1, 7).wait()


@functools.lru_cache(maxsize=1)
def _sc_fn():
    return pl.kernel(
        _sc_body,
        mesh=plsc.VectorSubcoreMesh(core_axis_name="c", subcore_axis_name="s"),
        out_type=jax.ShapeDtypeStruct((_SEQ * _NVT, _NBT, 1024), jnp.float32),
        scratch_types=[
            pltpu.VMEM((_TOK_W,), jnp.int32),
            pltpu.VMEM((128, 128), jnp.float32),
            pltpu.VMEM((128, 128), jnp.float32),
            pltpu.VMEM((16, 1024), jnp.float32),
            pltpu.VMEM((16, 1024), jnp.float32),
            pltpu.SemaphoreType.DMA,
            pltpu.SemaphoreType.DMA,
            pltpu.SemaphoreType.DMA,
            pltpu.SemaphoreType.DMA,
        ],
        compiler_params=pltpu.CompilerParams(use_tc_tiling_on_sc=False,
                                             needs_layout_passes=False),
    )


def kernel(x, emb_table, W, b):
    mc = _make_table(emb_table, W, b)
    xt = x.T.reshape(-1).astype(jnp.int32)
    out3 = _sc_fn()(mc, xt)
    out5 = out3.reshape(_SEQ, _NVT, _NBT, 8, 128)
    return out5.transpose(2, 4, 0, 1, 3).reshape(_BATCH, _SEQ, NUM_CHARS)


# speedup vs baseline: 2.8180x; 2.8180x over previous
import functools

import jax
import jax.numpy as jnp
from jax import lax
from jax.experimental import pallas as pl
from jax.experimental.pallas import tpu as pltpu
from jax.experimental.pallas import tpu_sc as plsc

NUM_CHARS = 1000
EMB_DIM = 64
_VPAD = 1024
_NCHUNK = 8
_BATCH = 4096
_SEQ = 20
_NVT = NUM_CHARS // 8


def _mm_body(emb_ref, w_ref, b_ref, mc_ref):
    mc_ref[0] = lax.dot_general(
        emb_ref[...], w_ref[0],
        dimension_numbers=(((1,), (1,)), ((), ())),
        preferred_element_type=jnp.float32,
    ) + b_ref[0]


def _make_table(emb_table, W, b):
    w_pad = jnp.zeros((_VPAD, EMB_DIM), jnp.float32).at[:NUM_CHARS].set(W)
    b_pad = jnp.zeros((_VPAD,), jnp.float32).at[:NUM_CHARS].set(b)
    return pl.pallas_call(
        _mm_body,
        grid=(_NCHUNK,),
        in_specs=[
            pl.BlockSpec((NUM_CHARS, EMB_DIM), lambda c: (0, 0)),
            pl.BlockSpec((1, 128, EMB_DIM), lambda c: (c, 0, 0)),
            pl.BlockSpec((1, 1, 128), lambda c: (c, 0, 0)),
        ],
        out_specs=pl.BlockSpec((1, NUM_CHARS, 128), lambda c: (c, 0, 0)),
        out_shape=jax.ShapeDtypeStruct((_NCHUNK, NUM_CHARS, 128), jnp.float32),
    )(emb_table, w_pad.reshape(_NCHUNK, 128, EMB_DIM),
      b_pad.reshape(_NCHUNK, 1, 128))


_NC, _NS = 2, 16
_NW = _NC * _NS
_NBT = _BATCH // 128
_NBLK = _SEQ * _NBT
_BLK_W = _NBLK // _NW
_TOK_W = _BLK_W * 128


def _sc_body(mc_hbm, xt_hbm, out_hbm, idx_v, g0, g1, t0, t1,
             sg0, sg1, sw0, sw1):
    wid = lax.axis_index("s") * _NC + lax.axis_index("c")
    tbase = wid * _TOK_W
    pltpu.sync_copy(xt_hbm.at[pl.ds(tbase, _TOK_W)], idx_v)

    G = (g0, g1)
    T = (t0, t1)
    SG = (sg0, sg1)
    SW = (sw0, sw1)
    iota = lax.iota(jnp.int32, 16)
    K = [(iota + d) & 15 for d in range(16)]
    RS = [k >> 3 for k in K]
    CS = [((k & 7) << 7) + iota for k in K]

    def g_desc(par, i_blk, c):
        return pltpu.make_async_copy(
            mc_hbm.at[c].at[idx_v.at[pl.ds(i_blk * 128, 128)]], G[par],
            SG[par])

    def w_desc(par, i_blk, c):
        blockid = wid * _BLK_W + i_blk
        s = blockid // _NBT
        bt = blockid % _NBT
        nvt = 16 if c < 7 else 13
        return pltpu.make_async_copy(
            T[par].at[pl.ds(0, nvt)],
            out_hbm.at[pl.ds(s * _NVT + 16 * c, nvt), bt],
            SW[par])

    def run_task(i_blk, c):
        par = c % 2
        g_desc(par, i_blk, c).wait()
        if c < 7:
            g_desc(1 - par, i_blk, c + 1).start()
        else:
            @pl.when(i_blk < _BLK_W - 1)
            def _():
                g_desc(1 - par, i_blk + 1, 0).start()
        if c >= 2:
            w_desc(par, i_blk, c - 2).wait()
        else:
            @pl.when(i_blk > 0)
            def _():
                w_desc(par, i_blk - 1, c + 6).wait()
        def trans_block(sb, carry):
            g = sb % 8
            vb = sb // 8
            sg = jnp.full((16,), 16 * g, jnp.int32)
            svb = jnp.full((16,), 16 * vb, jnp.int32)
            s2vb = jnp.full((16,), 2 * vb, jnp.int32)
            row_l = iota + sg
            for d in range(16):
                vals = plsc.load_gather(G[par], [row_l, K[d] + svb])
                plsc.store_scatter(T[par], [RS[d] + s2vb, CS[d] + sg], vals)
            return carry

        lax.fori_loop(0, 64, trans_block, 0)
        w_desc(par, i_blk, c).start()

    g_desc(0, 0, 0).start()

    def blk(i_blk, carry):
        for c in range(_NCHUNK):
            run_task(i_blk, c)
        return carry

    lax.fori_loop(0, _BLK_W, blk, 0)
    w_desc(0, _BLK_W - 1, 6).wait()
    w_desc(1, _BLK_W - 1, 7).wait()


@functools.lru_cache(maxsize=1)
def _sc_fn():
    return pl.kernel(
        _sc_body,
        mesh=plsc.VectorSubcoreMesh(core_axis_name="c", subcore_axis_name="s"),
        out_type=jax.ShapeDtypeStruct((_SEQ * _NVT, _NBT, 1024), jnp.float32),
        scratch_types=[
            pltpu.VMEM((_TOK_W,), jnp.int32),
            pltpu.VMEM((128, 128), jnp.float32),
            pltpu.VMEM((128, 128), jnp.float32),
            pltpu.VMEM((16, 1024), jnp.float32),
            pltpu.VMEM((16, 1024), jnp.float32),
            pltpu.SemaphoreType.DMA,
            pltpu.SemaphoreType.DMA,
            pltpu.SemaphoreType.DMA,
            pltpu.SemaphoreType.DMA,
        ],
        compiler_params=pltpu.CompilerParams(use_tc_tiling_on_sc=False,
                                             needs_layout_passes=False),
    )


def kernel(x, emb_table, W, b):
    mc = _make_table(emb_table, W, b)
    xt = x.T.reshape(-1).astype(jnp.int32)
    out3 = _sc_fn()(mc, xt)
    out5 = out3.reshape(_SEQ, _NVT, _NBT, 8, 128)
    return out5.transpose(2, 4, 0, 1, 3).reshape(_BATCH, _SEQ, NUM_CHARS)
